# baseline (speedup 1.0000x reference)
import functools

import numpy as np
import jax
import jax.numpy as jnp
from jax import lax
from jax.experimental import pallas as pl
from jax.experimental.pallas import tpu as pltpu
from jax.experimental.pallas import tpu_sc as plsc

VOCAB = 100000
D_MODEL = 1024
MAX_LEN = 2048
BATCH = 4
SEQ = 2048

NC, NS = 2, 16
NW = NC * NS
TOTAL = BATCH * SEQ
PER_W = TOTAL // NW
CHUNK = 16
N_CHUNKS = PER_W // CHUNK
NBUF = 4
GLEAD = 2
SCALE = float(D_MODEL) ** 0.5
PE_WORDS = D_MODEL // 2


def _make_pe(max_len, d_model):
    pe = np.zeros((max_len, d_model), dtype=np.float32)
    position = np.arange(0, max_len, dtype=np.float32)[:, None]
    div_term = np.exp(
        np.arange(0, d_model, 2, dtype=np.float32) * -(np.log(10000.0) / d_model))
    pe[:, 0::2] = np.sin(position * div_term)
    pe[:, 1::2] = np.cos(position * div_term)
    return pe


def _pack_pe(pe):
    bits = pe.view(np.uint32)
    bf = ((bits + 0x7FFF + ((bits >> 16) & 1)) >> 16).astype(np.uint32)
    g = bf.reshape(pe.shape[0], D_MODEL // 32, 2, 16)
    words = g[:, :, 0, :] | (g[:, :, 1, :] << 16)
    return words.reshape(pe.shape[0], PE_WORDS).view(np.int32)


_PE_PACKED = _pack_pe(_make_pe(MAX_LEN, D_MODEL))


def _sc_embed(x_flat, table, pe_pk):
    mesh = plsc.VectorSubcoreMesh(core_axis_name="c", subcore_axis_name="s")

    @functools.partial(
        pl.kernel,
        out_type=jax.ShapeDtypeStruct((TOTAL, D_MODEL), jnp.float32),
        mesh=mesh,
        scratch_types=[
            pltpu.VMEM((PER_W,), jnp.int32),
            [pltpu.VMEM((CHUNK, D_MODEL), jnp.float32) for _ in range(NBUF)],
            [pltpu.VMEM((CHUNK, PE_WORDS), jnp.int32) for _ in range(2)],
            [pltpu.SemaphoreType.DMA for _ in range(NBUF)],
            [pltpu.SemaphoreType.DMA for _ in range(2)],
            [pltpu.SemaphoreType.DMA for _ in range(NBUF)],
        ],
    )
    def k(x_hbm, table_hbm, pe_hbm, out_hbm,
          idx_v, rows, pebs, gsems, psems, osems):
        wid = lax.axis_index("s") * NC + lax.axis_index("c")
        base = wid * PER_W
        s0 = base % SEQ

        pltpu.sync_copy(x_hbm.at[pl.ds(base, PER_W)], idx_v)

        def fire_gather(c, b):
            pltpu.async_copy(
                table_hbm.at[idx_v.at[pl.ds(c * CHUNK, CHUNK)]],
                rows[b], gsems[b])

        def wait_gather(c, b):
            pltpu.make_async_copy(
                table_hbm.at[idx_v.at[pl.ds(c * CHUNK, CHUNK)]],
                rows[b], gsems[b]).wait()

        def fire_pe(c, b):
            pltpu.async_copy(
                pe_hbm.at[pl.ds(s0 + c * CHUNK, CHUNK)], pebs[b], psems[b])

        def wait_pe(c, b):
            pltpu.make_async_copy(
                pe_hbm.at[pl.ds(s0 + c * CHUNK, CHUNK)], pebs[b],
                psems[b]).wait()

        def fire_out(c, b):
            pltpu.async_copy(
                rows[b], out_hbm.at[pl.ds(base + c * CHUNK, CHUNK)], osems[b])

        def wait_out(c, b):
            pltpu.make_async_copy(
                rows[b], out_hbm.at[pl.ds(base + c * CHUNK, CHUNK)],
                osems[b]).wait()

        for c in range(GLEAD):
            fire_gather(c, c)
            fire_pe(c, c)

        def super_body(g, _):
            c0 = g * NBUF
            for j in range(NBUF):
                c = c0 + j
                bp = j % 2
                wait_gather(c, j)
                wait_pe(c, bp)

                @plsc.parallel_loop(0, CHUNK, unroll=2)
                def row_body(r):
                    for q in range(D_MODEL // 32):
                        w = pebs[bp][r, pl.ds(q * 16, 16)]
                        lo = lax.bitcast_convert_type(w << 16, jnp.float32)
                        hi = lax.bitcast_convert_type(
                            w & jnp.int32(-65536), jnp.float32)
                        sl0 = pl.ds(q * 32, 16)
                        sl1 = pl.ds(q * 32 + 16, 16)
                        rows[j][r, sl0] = rows[j][r, sl0] * SCALE + lo
                        rows[j][r, sl1] = rows[j][r, sl1] * SCALE + hi
                fire_out(c, j)

                @pl.when(c + GLEAD < N_CHUNKS)
                def _():
                    nb = (j + GLEAD) % NBUF

                    @pl.when(c >= GLEAD)
                    def _():
                        wait_out(c - GLEAD, nb)

                    fire_gather(c + GLEAD, nb)
                    fire_pe(c + GLEAD, bp)
            return 0

        lax.fori_loop(0, N_CHUNKS // NBUF, super_body, 0)

        for c in range(N_CHUNKS - GLEAD, N_CHUNKS):
            wait_out(c, c % NBUF)

    return k(x_flat, table, pe_pk)


def kernel(x, table):
    x_flat = jnp.reshape(x, (TOTAL,)).astype(jnp.int32)
    out = _sc_embed(x_flat, table, _PE_PACKED)
    return jnp.reshape(out, (BATCH, SEQ, D_MODEL))


# speedup vs baseline: 1.3114x; 1.3114x over previous
import functools

import numpy as np
import jax
import jax.numpy as jnp
from jax import lax
from jax.experimental import pallas as pl
from jax.experimental.pallas import tpu as pltpu
from jax.experimental.pallas import tpu_sc as plsc

VOCAB = 100000
D_MODEL = 1024
MAX_LEN = 2048
BATCH = 4
SEQ = 2048

NC, NS = 2, 16
NW = NC * NS
TOTAL = BATCH * SEQ
PER_W = TOTAL // NW
CHUNK = 8
N_CHUNKS = PER_W // CHUNK
NR = 8
NO = 4
GLEAD = 6
SCALE = float(D_MODEL) ** 0.5


def _make_pe(max_len, d_model):
    pe = np.zeros((max_len, d_model), dtype=np.float32)
    position = np.arange(0, max_len, dtype=np.float32)[:, None]
    div_term = np.exp(
        np.arange(0, d_model, 2, dtype=np.float32) * -(np.log(10000.0) / d_model))
    pe[:, 0::2] = np.sin(position * div_term)
    pe[:, 1::2] = np.cos(position * div_term)
    return pe


_PE = _make_pe(MAX_LEN, D_MODEL)


def _sc_embed(x_flat, table, pe):
    mesh = plsc.VectorSubcoreMesh(core_axis_name="c", subcore_axis_name="s")

    @functools.partial(
        pl.kernel,
        out_type=jax.ShapeDtypeStruct((TOTAL, D_MODEL), jnp.float32),
        mesh=mesh,
        scratch_types=[
            pltpu.VMEM((PER_W,), jnp.int32),
            [pltpu.VMEM((CHUNK, D_MODEL), jnp.float32) for _ in range(NR)],
            [pltpu.VMEM((CHUNK, D_MODEL), jnp.float32) for _ in range(NO)],
            [pltpu.SemaphoreType.DMA for _ in range(NR)],
            [pltpu.SemaphoreType.DMA for _ in range(NO)],
            [pltpu.SemaphoreType.DMA for _ in range(NO)],
        ],
    )
    def k(x_hbm, table_hbm, pe_hbm, out_hbm,
          idx_v, rows, obufs, gsems, psems, osems):
        wid = lax.axis_index("s") * NC + lax.axis_index("c")
        base = wid * PER_W
        s0 = base % SEQ

        pltpu.sync_copy(x_hbm.at[pl.ds(base, PER_W)], idx_v)

        def fire_gather(c, b):
            pltpu.async_copy(
                table_hbm.at[idx_v.at[pl.ds(c * CHUNK, CHUNK)]],
                rows[b], gsems[b])

        def wait_gather(c, b):
            pltpu.make_async_copy(
                table_hbm.at[idx_v.at[pl.ds(c * CHUNK, CHUNK)]],
                rows[b], gsems[b]).wait()

        def fire_pe(c, b):
            pltpu.async_copy(
                pe_hbm.at[pl.ds(s0 + c * CHUNK, CHUNK)], obufs[b], psems[b])

        def wait_pe(c, b):
            pltpu.make_async_copy(
                pe_hbm.at[pl.ds(s0 + c * CHUNK, CHUNK)], obufs[b],
                psems[b]).wait()

        def fire_out(c, b):
            pltpu.async_copy(
                obufs[b], out_hbm.at[pl.ds(base + c * CHUNK, CHUNK)], osems[b])

        def wait_out(c, b):
            pltpu.make_async_copy(
                obufs[b], out_hbm.at[pl.ds(base + c * CHUNK, CHUNK)],
                osems[b]).wait()

        for c in range(GLEAD):
            fire_gather(c, c)
        for c in range(2):
            fire_pe(c, c)

        def super_body(g, _):
            c0 = g * NR
            for j in range(NR):
                c = c0 + j
                bo = j % NO
                wait_gather(c, j)
                wait_pe(c, bo)

                @plsc.parallel_loop(0, CHUNK)
                def row_body(r):
                    for q in range(D_MODEL // 16):
                        sl = pl.ds(q * 16, 16)
                        plsc.addupdate(
                            obufs[bo].at[r, sl], rows[j][r, sl] * SCALE)

                fire_out(c, bo)

                bo2 = (j + 2) % NO

                @pl.when(c + 2 < N_CHUNKS)
                def _():
                    @pl.when(c >= 2)
                    def _():
                        wait_out(c - 2, bo2)

                    fire_pe(c + 2, bo2)

                @pl.when(c + GLEAD < N_CHUNKS)
                def _():
                    fire_gather(c + GLEAD, (j + GLEAD) % NR)
            return 0

        lax.fori_loop(0, N_CHUNKS // NR, super_body, 0)

        for c in range(N_CHUNKS - 2, N_CHUNKS):
            wait_out(c, c % NO)

    return k(x_flat, table, pe)


def kernel(x, table):
    x_flat = jnp.reshape(x, (TOTAL,)).astype(jnp.int32)
    out = _sc_embed(x_flat, table, _PE)
    return jnp.reshape(out, (BATCH, SEQ, D_MODEL))
